# Initial kernel scaffold; baseline (speedup 1.0000x reference)
import jax
import jax.numpy as jnp
from jax.experimental import pallas as pl


def kernel(x0, x1, x2):
    raise NotImplementedError("write your pallas kernel here")


# speedup vs baseline: 24.7365x; 24.7365x over previous
import numpy as np
import jax
import jax.numpy as jnp
from jax.experimental import pallas as pl

_NC = 80
_NB, _AR, _MR = 5, 14, 56
_ND = 100
_SEL0 = 100


def _bilinear_mat() -> np.ndarray:
    i = np.arange(_MR, dtype=np.float64)
    sample = (i + 0.5) * (_AR / _MR) - 0.5
    x = np.abs(sample[None, :] - np.arange(_AR, dtype=np.float64)[:, None])
    w = np.maximum(0.0, 1.0 - x)
    w = w / w.sum(axis=0, keepdims=True)
    return w


_W1 = _bilinear_mat()
_W2 = np.kron(_W1, _W1).astype(np.float32)


_POOLED = np.asarray(
    jax.random.normal(jax.random.key(1), (_ND, _NB, _MR, _MR), dtype=jnp.float32)
).transpose(1, 0, 2, 3).reshape(_NB, _ND, _MR * _MR).copy()


def _fused_kernel(a_ref, sa_ref, w2_ref, pooled_ref, out_ref):
    a = a_ref[...]
    conf = a[:, 4:5]
    scores = a[:, 5:] * conf
    mx = jnp.max(scores, axis=1, keepdims=True)
    cat = jnp.argmax(scores, axis=1).astype(jnp.float32)[:, None]
    b0, b1, b2, b3 = a[:, 0:1], a[:, 1:2], a[:, 2:3], a[:, 3:4]
    boxes = jnp.concatenate(
        [b0 - 0.5 * b2, b1 - 0.5 * b3, b0 + 0.5 * b2, b1 + 0.5 * b3], axis=1)

    r = jax.lax.dot_general(sa_ref[...], w2_ref[...],
                            (((2,), (0,)), ((), ())),
                            preferred_element_type=jnp.float32)
    m = jnp.max(r, axis=0)
    e = jnp.exp(r - m[None])
    s = jnp.sum(e, axis=0)
    num = jnp.sum(pooled_ref[...] * e, axis=0)
    masks = jax.nn.sigmoid(num / s)

    head = jnp.concatenate([jnp.zeros_like(mx), boxes, cat, mx], axis=1)
    out_ref[...] = jnp.concatenate([head, masks], axis=1)


def kernel(x0, x1, x2):
    del x2
    a = x0[0, _SEL0:_SEL0 + _ND, :]
    sa = x1[0, _SEL0:_SEL0 + _ND, :].reshape(_ND, _NB, _AR * _AR).transpose(1, 0, 2)
    return pl.pallas_call(
        _fused_kernel,
        out_shape=jax.ShapeDtypeStruct((_ND, 7 + _MR * _MR), jnp.float32),
    )(a, sa, jnp.asarray(_W2), jnp.asarray(_POOLED))
